# Initial kernel scaffold; baseline (speedup 1.0000x reference)
import jax
import jax.numpy as jnp
from jax.experimental import pallas as pl


def kernel(x, pos, edge_index, h_w1, h_b1, h_w2, h_b2, f_w1, f_b1, f_w2, f_b2, g_w1, g_b1, g_w2, g_b2):
    raise NotImplementedError("write your pallas kernel here")


# speedup vs baseline: 12.1778x; 12.1778x over previous
import functools

import jax
import jax.numpy as jnp
from jax.experimental import pallas as pl

_RATIO = 0.25
_NEG_INF = float("-inf")


def _fps_body(px_ref, py_ref, pz_ref, idx_ref, cx_ref, cy_ref, cz_ref, *, n, m):
    px = px_ref[...]
    py = py_ref[...]
    pz = pz_ref[...]
    rows, cols = px.shape
    niota = (
        jax.lax.broadcasted_iota(jnp.int32, (rows, cols), 0) * cols
        + jax.lax.broadcasted_iota(jnp.int32, (rows, cols), 1)
    )
    orows, ocols = idx_ref.shape
    miota = (
        jax.lax.broadcasted_iota(jnp.int32, (orows, ocols), 0) * ocols
        + jax.lax.broadcasted_iota(jnp.int32, (orows, ocols), 1)
    )

    sel0 = niota == 0
    zf = jnp.zeros_like(px)
    x0 = jnp.sum(jnp.where(sel0, px, zf))
    y0 = jnp.sum(jnp.where(sel0, py, zf))
    z0 = jnp.sum(jnp.where(sel0, pz, zf))

    dx = px - x0
    dy = py - y0
    dz = pz - z0
    md = (dx * dx + dy * dy) + dz * dz
    md = jnp.where(niota < n, md, _NEG_INF)

    idx_ref[...] = jnp.zeros((orows, ocols), jnp.int32)
    osel0 = miota == 0
    ozf = jnp.zeros((orows, ocols), jnp.float32)
    cx_ref[...] = jnp.where(osel0, x0, ozf)
    cy_ref[...] = jnp.where(osel0, y0, ozf)
    cz_ref[...] = jnp.where(osel0, z0, ozf)

    def body(i, md):
        mval = jnp.max(md)
        nxt = jnp.min(jnp.where(md == mval, niota, jnp.int32(2**30)))
        selm = niota == nxt
        sx = jnp.sum(jnp.where(selm, px, zf))
        sy = jnp.sum(jnp.where(selm, py, zf))
        sz = jnp.sum(jnp.where(selm, pz, zf))
        ddx = px - sx
        ddy = py - sy
        ddz = pz - sz
        dn = (ddx * ddx + ddy * ddy) + ddz * ddz
        md = jnp.minimum(md, dn)
        md = jnp.where(niota < n, md, _NEG_INF)
        seli = miota == i
        idx_ref[...] = jnp.where(seli, nxt, idx_ref[...])
        cx_ref[...] = jnp.where(seli, sx, cx_ref[...])
        cy_ref[...] = jnp.where(seli, sy, cy_ref[...])
        cz_ref[...] = jnp.where(seli, sz, cz_ref[...])
        return md

    jax.lax.fori_loop(1, m, body, md)


def _fps_pallas(pos, m, interpret=False):
    n = pos.shape[0]
    npad = ((n + 1279) // 1280) * 1280
    mpad = ((m + 319) // 320) * 320
    posp = jnp.pad(pos, ((0, npad - n), (0, 0)))
    px = posp[:, 0].reshape(8, npad // 8)
    py = posp[:, 1].reshape(8, npad // 8)
    pz = posp[:, 2].reshape(8, npad // 8)
    out_shape = [
        jax.ShapeDtypeStruct((8, mpad // 8), jnp.int32),
        jax.ShapeDtypeStruct((8, mpad // 8), jnp.float32),
        jax.ShapeDtypeStruct((8, mpad // 8), jnp.float32),
        jax.ShapeDtypeStruct((8, mpad // 8), jnp.float32),
    ]
    idx8, cx8, cy8, cz8 = pl.pallas_call(
        functools.partial(_fps_body, n=n, m=m),
        out_shape=out_shape,
        interpret=interpret,
    )(px, py, pz)
    idx = idx8.reshape(-1)[:m]
    pos_s = jnp.stack(
        [cx8.reshape(-1)[:m], cy8.reshape(-1)[:m], cz8.reshape(-1)[:m]], axis=1
    )
    return idx, pos_s


def _mlp2(x, w1, b1, w2, b2):
    return jnp.maximum(x @ w1 + b1, 0.0) @ w2 + b2


def kernel(x, pos, edge_index, h_w1, h_b1, h_w2, h_b2, f_w1, f_b1, f_w2, f_b2,
           g_w1, g_b1, g_w2, g_b2):
    n = x.shape[0]
    m = int(round(n * _RATIO))
    idx, pos_s = _fps_pallas(pos, m)

    d2 = (
        jnp.sum(pos**2, axis=1)[:, None]
        + jnp.sum(pos_s**2, axis=1)[None, :]
        - 2.0 * pos @ pos_s.T
    )
    s = jnp.argmin(d2, axis=1).astype(jnp.int32)
    q = jnp.arange(n, dtype=jnp.int32)
    assign_index = jnp.stack([q, s])
    x_dst = x[idx]
    dist = pos - pos_s[s]
    delta_dst = _mlp2(x_dst, h_w1, h_b1, h_w2, h_b2)
    delta = delta_dst[s]
    msg = _mlp2(jnp.concatenate([dist + delta, x], axis=-1), f_w1, f_b1, f_w2, f_b2)
    agg = jax.ops.segment_max(msg, s, num_segments=m)
    agg = jnp.where(jnp.isfinite(agg), agg, 0.0)
    out = _mlp2(jnp.concatenate([x_dst, agg], axis=-1), g_w1, g_b1, g_w2, g_b2)
    return (x_dst + out, pos_s, assign_index)


# speedup vs baseline: 12.7391x; 1.0461x over previous
import functools

import jax
import jax.numpy as jnp
from jax import lax
from jax.experimental import pallas as pl
from jax.experimental.pallas import tpu as pltpu
from jax.experimental.pallas import tpu_sc as plsc

_RATIO = 0.25
_NEG_INF = float("-inf")


def _fps_body(px_ref, py_ref, pz_ref, idx_ref, cx_ref, cy_ref, cz_ref, cn_ref,
              *, n, m):
    px = px_ref[...]
    py = py_ref[...]
    pz = pz_ref[...]
    rows, cols = px.shape
    niota = (
        lax.broadcasted_iota(jnp.int32, (rows, cols), 0) * cols
        + lax.broadcasted_iota(jnp.int32, (rows, cols), 1)
    )
    orows, ocols = idx_ref.shape
    miota = (
        lax.broadcasted_iota(jnp.int32, (orows, ocols), 0) * ocols
        + lax.broadcasted_iota(jnp.int32, (orows, ocols), 1)
    )

    sel0 = niota == 0
    zf = jnp.zeros_like(px)
    x0 = jnp.sum(jnp.where(sel0, px, zf))
    y0 = jnp.sum(jnp.where(sel0, py, zf))
    z0 = jnp.sum(jnp.where(sel0, pz, zf))

    dx = px - x0
    dy = py - y0
    dz = pz - z0
    md = (dx * dx + dy * dy) + dz * dz
    md = jnp.where(niota < n, md, _NEG_INF)

    idx_ref[...] = jnp.zeros((orows, ocols), jnp.int32)
    osel0 = miota == 0
    ozf = jnp.zeros((orows, ocols), jnp.float32)
    cx_ref[...] = jnp.where(osel0, x0, ozf)
    cy_ref[...] = jnp.where(osel0, y0, ozf)
    cz_ref[...] = jnp.where(osel0, z0, ozf)

    def body(i, md):
        mval = jnp.max(md)
        nxt = jnp.min(jnp.where(md == mval, niota, jnp.int32(2**30)))
        selm = niota == nxt
        sx = jnp.sum(jnp.where(selm, px, zf))
        sy = jnp.sum(jnp.where(selm, py, zf))
        sz = jnp.sum(jnp.where(selm, pz, zf))
        ddx = px - sx
        ddy = py - sy
        ddz = pz - sz
        dn = (ddx * ddx + ddy * ddy) + ddz * ddz
        md = jnp.minimum(md, dn)
        md = jnp.where(niota < n, md, _NEG_INF)
        seli = miota == i
        idx_ref[...] = jnp.where(seli, nxt, idx_ref[...])
        cx_ref[...] = jnp.where(seli, sx, cx_ref[...])
        cy_ref[...] = jnp.where(seli, sy, cy_ref[...])
        cz_ref[...] = jnp.where(seli, sz, cz_ref[...])
        return md

    lax.fori_loop(1, m, body, md)

    cx = cx_ref[...]
    cy = cy_ref[...]
    cz = cz_ref[...]
    cn = (cx * cx + cy * cy) + cz * cz
    cn_ref[...] = jnp.where(miota < m, cn, jnp.float32(1e30))


def _fps_pallas(pos, m, interpret=False):
    n = pos.shape[0]
    npad = ((n + 1279) // 1280) * 1280
    mpad = ((m + 319) // 320) * 320
    posp = jnp.pad(pos, ((0, npad - n), (0, 0)))
    px = posp[:, 0].reshape(8, npad // 8)
    py = posp[:, 1].reshape(8, npad // 8)
    pz = posp[:, 2].reshape(8, npad // 8)
    sh = (8, mpad // 8)
    out_shape = [
        jax.ShapeDtypeStruct(sh, jnp.int32),
        jax.ShapeDtypeStruct(sh, jnp.float32),
        jax.ShapeDtypeStruct(sh, jnp.float32),
        jax.ShapeDtypeStruct(sh, jnp.float32),
        jax.ShapeDtypeStruct(sh, jnp.float32),
    ]
    return pl.pallas_call(
        functools.partial(_fps_body, n=n, m=m),
        out_shape=out_shape,
        interpret=interpret,
    )(px, py, pz)


def _sc_gather(table, idx, bpad):
    nw = 32
    bw = bpad // nw
    mesh = plsc.VectorSubcoreMesh(core_axis_name="c", subcore_axis_name="s")

    @functools.partial(
        pl.kernel,
        mesh=mesh,
        out_type=jax.ShapeDtypeStruct((bpad, table.shape[1]), jnp.float32),
        scratch_types=[
            pltpu.VMEM((bw,), jnp.int32),
            pltpu.VMEM((bw, table.shape[1]), jnp.float32),
            pltpu.SemaphoreType.DMA,
        ],
    )
    def gk(table_hbm, idx_hbm, out_hbm, idxv, rowsv, sem):
        wid = lax.axis_index("s") * 2 + lax.axis_index("c")
        base = wid * bw
        pltpu.sync_copy(idx_hbm.at[pl.ds(base, bw)], idxv)
        pltpu.async_copy(table_hbm.at[idxv], rowsv, sem).wait()
        pltpu.sync_copy(rowsv, out_hbm.at[pl.ds(base, bw)])

    return gk(table, idx)


def _mlph_body(xd_ref, ps_ref, w1_ref, b1_ref, w2_ref, b2_ref, e_ref):
    t = jnp.maximum(
        jnp.dot(xd_ref[...], w1_ref[...], preferred_element_type=jnp.float32)
        + b1_ref[...],
        0.0,
    )
    delta = (
        jnp.dot(t, w2_ref[...], preferred_element_type=jnp.float32) + b2_ref[...]
    )
    e_ref[...] = ps_ref[...] - delta


def _mlph_pallas(x_dst, pos_s, w1, b1, w2, b2):
    m = x_dst.shape[0]
    return pl.pallas_call(
        _mlph_body,
        out_shape=jax.ShapeDtypeStruct((m, 3), jnp.float32),
    )(x_dst, pos_s, w1, b1.reshape(1, -1), w2, b2.reshape(1, -1))


def _knn_msg_body(p8_ref, x_ref, ct_ref, cn_ref, e8_ref, wx_ref, wd8_ref,
                  w2_ref, b2_ref, s_ref, msg_ref):
    p8 = p8_ref[...]
    r = jnp.dot(p8, ct_ref[...], preferred_element_type=jnp.float32)
    p0 = p8[:, 0:1]
    p1 = p8[:, 1:2]
    p2 = p8[:, 2:3]
    pn = (p0 * p0 + p1 * p1) + p2 * p2
    d2 = (pn + cn_ref[...]) - 2.0 * r
    mn = jnp.min(d2, axis=1, keepdims=True)
    liota = lax.broadcasted_iota(jnp.int32, d2.shape, 1)
    sblk = jnp.min(
        jnp.where(d2 == mn, liota, jnp.int32(2**30)), axis=1, keepdims=True
    )
    s_ref[...] = sblk.reshape(1, 1, sblk.shape[0])
    oh = jnp.where(liota == sblk, 1.0, 0.0).astype(jnp.float32)
    u8 = p8 - jnp.dot(oh, e8_ref[...], preferred_element_type=jnp.float32)
    t = jnp.maximum(
        jnp.dot(x_ref[...], wx_ref[...], preferred_element_type=jnp.float32)
        + jnp.dot(u8, wd8_ref[...], preferred_element_type=jnp.float32),
        0.0,
    )
    msg_ref[...] = (
        jnp.dot(t, w2_ref[...], preferred_element_type=jnp.float32) + b2_ref[...]
    )


def _knn_msg_pallas(p8, x, ct8, cn, e8, wx, wd8, w2, b2, bn=400):
    n, d = x.shape
    grid = n // bn
    mp = ct8.shape[1]
    s3, msg = pl.pallas_call(
        _knn_msg_body,
        grid=(grid,),
        in_specs=[
            pl.BlockSpec((bn, 8), lambda i: (i, 0)),
            pl.BlockSpec((bn, d), lambda i: (i, 0)),
            pl.BlockSpec((8, mp), lambda i: (0, 0)),
            pl.BlockSpec((1, mp), lambda i: (0, 0)),
            pl.BlockSpec((mp, 8), lambda i: (0, 0)),
            pl.BlockSpec((d, d), lambda i: (0, 0)),
            pl.BlockSpec((8, d), lambda i: (0, 0)),
            pl.BlockSpec((d, d), lambda i: (0, 0)),
            pl.BlockSpec((1, d), lambda i: (0, 0)),
        ],
        out_specs=[
            pl.BlockSpec((1, 1, bn), lambda i: (i, 0, 0)),
            pl.BlockSpec((bn, d), lambda i: (i, 0)),
        ],
        out_shape=[
            jax.ShapeDtypeStruct((grid, 1, bn), jnp.int32),
            jax.ShapeDtypeStruct((n, d), jnp.float32),
        ],
    )(p8, x, ct8, cn, e8, wx, wd8, w2, b2)
    return s3.reshape(n), msg


def _sc_segmax(msg, s, m):
    n, d = msg.shape
    rpg = -(-(n // 4) // 8) * 8
    starts = [0] + [((n // 4) * g // 8) * 8 for g in (1, 2, 3)]
    assert all(st + rpg <= n for st in starts) and (starts[3] + rpg == n)
    mesh = plsc.VectorSubcoreMesh(core_axis_name="c", subcore_axis_name="s")

    @functools.partial(
        pl.kernel,
        mesh=mesh,
        out_type=jax.ShapeDtypeStruct((2, m, d), jnp.float32),
        compiler_params=pltpu.CompilerParams(use_tc_tiling_on_sc=False),
        scratch_types=[
            pltpu.VMEM((m, 16), jnp.float32),
            pltpu.VMEM((rpg, 16), jnp.float32),
            pltpu.VMEM((rpg,), jnp.int32),
            pltpu.VMEM_SHARED((8, m, 16), jnp.float32),
        ],
    )
    def sk(msg_hbm, s_hbm, out_hbm, acc, msgv, sv, shared):
        c = lax.axis_index("c")
        sc = lax.axis_index("s")
        cg = sc % 8
        rgl = sc // 8
        col0 = cg * 16
        rg = c * 2 + rgl
        st = (rg * (n // 4)) // 8 * 8
        row0 = pl.multiple_of(st, 8)
        pltpu.sync_copy(s_hbm.at[pl.ds(row0, rpg)], sv)
        pltpu.sync_copy(msg_hbm.at[pl.ds(row0, rpg), pl.ds(col0, 16)], msgv)
        ninf = jnp.full((16,), _NEG_INF, jnp.float32)

        def init(i, _):
            acc[i] = ninf
            return 0

        lax.fori_loop(0, m, init, 0, unroll=8)

        nch = (rpg + 15) // 16
        last = ((rpg - 16) // 8) * 8

        def body(ch, _):
            base = pl.multiple_of(jnp.minimum(ch * 16, last), 8)
            svec = sv[pl.ds(base, 16)]
            for k in range(16):
                si = svec[k]
                acc[si] = jnp.maximum(acc[si], msgv[base + k])
            return 0

        lax.fori_loop(0, nch, body, 0)

        @pl.when(rgl == 1)
        def _():
            pltpu.sync_copy(acc, shared.at[cg])

        plsc.subcore_barrier()

        @pl.when(rgl == 0)
        def _():
            pltpu.sync_copy(shared.at[cg], msgv.at[pl.ds(0, m), :])

            def comb(i, _):
                acc[i] = jnp.maximum(acc[i], msgv[i])
                return 0

            lax.fori_loop(0, m, comb, 0, unroll=8)
            pltpu.sync_copy(acc, out_hbm.at[c, :, pl.ds(col0, 16)])

    return sk(msg, s)


def _mlpg_body(xd_ref, pa_ref, pb_ref, w1a_ref, w1b_ref, b1_ref, w2_ref,
               b2_ref, out_ref):
    agg = jnp.maximum(pa_ref[...], pb_ref[...])
    agg = jnp.where(agg == _NEG_INF, 0.0, agg)
    xd = xd_ref[...]
    t = jnp.maximum(
        jnp.dot(xd, w1a_ref[...], preferred_element_type=jnp.float32)
        + jnp.dot(agg, w1b_ref[...], preferred_element_type=jnp.float32)
        + b1_ref[...],
        0.0,
    )
    out_ref[...] = (
        xd
        + jnp.dot(t, w2_ref[...], preferred_element_type=jnp.float32)
        + b2_ref[...]
    )


def _mlpg_pallas(x_dst, parts, w1a, w1b, b1, w2, b2):
    m, d = x_dst.shape
    return pl.pallas_call(
        _mlpg_body,
        out_shape=jax.ShapeDtypeStruct((m, d), jnp.float32),
    )(x_dst, parts[0], parts[1], w1a, w1b, b1.reshape(1, -1), w2,
      b2.reshape(1, -1))


def kernel(x, pos, edge_index, h_w1, h_b1, h_w2, h_b2, f_w1, f_b1, f_w2, f_b2,
           g_w1, g_b1, g_w2, g_b2):
    n, d = x.shape
    m = int(round(n * _RATIO))
    mp = 2560

    idx8, cx8, cy8, cz8, cn8 = _fps_pallas(pos, m)
    idx = idx8.reshape(-1)[:m]
    cxr = cx8.reshape(1, -1)
    cyr = cy8.reshape(1, -1)
    czr = cz8.reshape(1, -1)
    pos_s = jnp.concatenate([cxr, cyr, czr], axis=0).T[:m]

    ct8 = jnp.concatenate(
        [cxr, cyr, czr, jnp.zeros((5, mp), jnp.float32)], axis=0)
    cn = cn8.reshape(1, -1)

    idx_p = jnp.pad(idx, (0, mp - m))
    x_dst_p = _sc_gather(x, idx_p, mp)
    x_dst = x_dst_p[:m]

    e3 = _mlph_pallas(x_dst, pos_s, h_w1, h_b1, h_w2, h_b2)
    e8 = jnp.pad(e3, ((0, mp - m), (0, 5)))

    p8 = jnp.concatenate(
        [pos, jnp.ones((n, 1), jnp.float32), jnp.zeros((n, 4), jnp.float32)],
        axis=1)
    wd8 = jnp.concatenate([f_w1[:3], f_b1.reshape(1, -1),
                           jnp.zeros((4, d), jnp.float32)], axis=0)

    s, msg = _knn_msg_pallas(p8, x, ct8, cn, e8, f_w1[3:], wd8, f_w2,
                             f_b2.reshape(1, -1))

    parts = _sc_segmax(msg, s, m)

    out1 = _mlpg_pallas(x_dst, parts, g_w1[:d], g_w1[d:], g_b1, g_w2, g_b2)

    q = jnp.arange(n, dtype=jnp.int32)
    assign_index = jnp.stack([q, s])
    return (out1, pos_s, assign_index)


# speedup vs baseline: 22.8737x; 1.7956x over previous
import functools

import jax
import jax.numpy as jnp
from jax import lax
from jax.experimental import pallas as pl
from jax.experimental.pallas import tpu as pltpu
from jax.experimental.pallas import tpu_sc as plsc

_RATIO = 0.25
_NEG_INF = float("-inf")


def _fps_body(px_ref, py_ref, pz_ref, idx_ref, cx_ref, cy_ref, cz_ref, cn_ref,
              fi_ref, *, n, m):
    px = px_ref[...]
    py = py_ref[...]
    pz = pz_ref[...]
    rows, cols = px.shape
    niota = (
        lax.broadcasted_iota(jnp.int32, (rows, cols), 0) * cols
        + lax.broadcasted_iota(jnp.int32, (rows, cols), 1)
    )
    orows, ocols = idx_ref.shape
    miota = (
        lax.broadcasted_iota(jnp.int32, (orows, ocols), 0) * ocols
        + lax.broadcasted_iota(jnp.int32, (orows, ocols), 1)
    )

    sel0 = niota == 0
    zf = jnp.zeros_like(px)
    x0 = jnp.sum(jnp.where(sel0, px, zf))
    y0 = jnp.sum(jnp.where(sel0, py, zf))
    z0 = jnp.sum(jnp.where(sel0, pz, zf))

    dx = px - x0
    dy = py - y0
    dz = pz - z0
    md = (dx * dx + dy * dy) + dz * dz
    md = jnp.where(niota < n, md, _NEG_INF)

    idx_ref[...] = jnp.zeros((orows, ocols), jnp.int32)
    osel0 = miota == 0
    ozf = jnp.zeros((orows, ocols), jnp.float32)
    cx_ref[...] = jnp.where(osel0, x0, ozf)
    cy_ref[...] = jnp.where(osel0, y0, ozf)
    cz_ref[...] = jnp.where(osel0, z0, ozf)

    def _amax2(a):
        return jnp.max(jnp.max(a, axis=1, keepdims=True), axis=0,
                       keepdims=True)

    def _amin2(a):
        return jnp.min(jnp.min(a, axis=1, keepdims=True), axis=0,
                       keepdims=True)

    def _asum2(a):
        return jnp.sum(jnp.sum(a, axis=1, keepdims=True), axis=0,
                       keepdims=True)

    fi_ref[...] = niota.astype(jnp.float32)

    def body(i, md):
        mval = _amax2(md)
        selv = md == mval
        fidx = jnp.where(selv, fi_ref[...], jnp.float32(3.0e7))
        nxtf = _amin2(fidx)
        sxf = _asum2(jnp.where(selv, px_ref[...], 0.0))
        syf = _asum2(jnp.where(selv, py_ref[...], 0.0))
        szf = _asum2(jnp.where(selv, pz_ref[...], 0.0))
        cnt = _asum2(jnp.where(selv, 1.0, 0.0))

        ddx = px_ref[...] - sxf
        ddy = py_ref[...] - syf
        ddz = pz_ref[...] - szf
        dn = (ddx * ddx + ddy * ddy) + ddz * ddz
        md_fast = jnp.minimum(md, dn)

        def tiecase(_):
            selm = fi_ref[...] == nxtf
            sx = _asum2(jnp.where(selm, px_ref[...], 0.0))
            sy = _asum2(jnp.where(selm, py_ref[...], 0.0))
            sz = _asum2(jnp.where(selm, pz_ref[...], 0.0))
            tx = px_ref[...] - sx
            ty = py_ref[...] - sy
            tz = pz_ref[...] - sz
            tn = (tx * tx + ty * ty) + tz * tz
            return jnp.minimum(md, tn), sx, sy, sz

        md, sx, sy, sz = lax.cond(
            cnt[0, 0] > 1.0, tiecase,
            lambda _: (md_fast, sxf, syf, szf), None)
        nxt = nxtf.astype(jnp.int32)
        seli = miota == i
        idx_ref[...] = jnp.where(seli, nxt, idx_ref[...])
        cx_ref[...] = jnp.where(seli, sx, cx_ref[...])
        cy_ref[...] = jnp.where(seli, sy, cy_ref[...])
        cz_ref[...] = jnp.where(seli, sz, cz_ref[...])
        return md

    lax.fori_loop(1, m, body, md, unroll=2)

    cx = cx_ref[...]
    cy = cy_ref[...]
    cz = cz_ref[...]
    cn = (cx * cx + cy * cy) + cz * cz
    cn_ref[...] = jnp.where(miota < m, cn, jnp.float32(1e30))


def _fps_pallas(pos, m, interpret=False):
    n = pos.shape[0]
    npad = ((n + 1279) // 1280) * 1280
    mpad = ((m + 319) // 320) * 320
    posp = jnp.pad(pos, ((0, npad - n), (0, 0)))
    px = posp[:, 0].reshape(8, npad // 8)
    py = posp[:, 1].reshape(8, npad // 8)
    pz = posp[:, 2].reshape(8, npad // 8)
    sh = (8, mpad // 8)
    out_shape = [
        jax.ShapeDtypeStruct(sh, jnp.int32),
        jax.ShapeDtypeStruct(sh, jnp.float32),
        jax.ShapeDtypeStruct(sh, jnp.float32),
        jax.ShapeDtypeStruct(sh, jnp.float32),
        jax.ShapeDtypeStruct(sh, jnp.float32),
    ]
    return pl.pallas_call(
        functools.partial(_fps_body, n=n, m=m),
        out_shape=out_shape,
        scratch_shapes=[pltpu.VMEM((8, npad // 8), jnp.float32)],
        interpret=interpret,
    )(px, py, pz)


def _sc_gather(table, idx, bpad):
    nw = 32
    bw = bpad // nw
    mesh = plsc.VectorSubcoreMesh(core_axis_name="c", subcore_axis_name="s")

    @functools.partial(
        pl.kernel,
        mesh=mesh,
        out_type=jax.ShapeDtypeStruct((bpad, table.shape[1]), jnp.float32),
        scratch_types=[
            pltpu.VMEM((bw,), jnp.int32),
            pltpu.VMEM((bw, table.shape[1]), jnp.float32),
            pltpu.SemaphoreType.DMA,
        ],
    )
    def gk(table_hbm, idx_hbm, out_hbm, idxv, rowsv, sem):
        wid = lax.axis_index("s") * 2 + lax.axis_index("c")
        base = wid * bw
        pltpu.sync_copy(idx_hbm.at[pl.ds(base, bw)], idxv)
        pltpu.async_copy(table_hbm.at[idxv], rowsv, sem).wait()
        pltpu.sync_copy(rowsv, out_hbm.at[pl.ds(base, bw)])

    return gk(table, idx)


def _mlph_body(xd_ref, ps_ref, w1_ref, b1_ref, w2_ref, b2_ref, e_ref):
    t = jnp.maximum(
        jnp.dot(xd_ref[...], w1_ref[...], preferred_element_type=jnp.float32)
        + b1_ref[...],
        0.0,
    )
    delta = (
        jnp.dot(t, w2_ref[...], preferred_element_type=jnp.float32) + b2_ref[...]
    )
    e_ref[...] = ps_ref[...] - delta


def _mlph_pallas(x_dst, pos_s, w1, b1, w2, b2):
    m = x_dst.shape[0]
    return pl.pallas_call(
        _mlph_body,
        out_shape=jax.ShapeDtypeStruct((m, 3), jnp.float32),
    )(x_dst, pos_s, w1, b1.reshape(1, -1), w2, b2.reshape(1, -1))


def _knn_msg_body(p8_ref, x_ref, ct_ref, cn_ref, e8_ref, wx_ref, wd8_ref,
                  w2_ref, b2_ref, s_ref, msg_ref):
    p8 = p8_ref[...]
    r = jnp.dot(p8, ct_ref[...], preferred_element_type=jnp.float32)
    p0 = p8[:, 0:1]
    p1 = p8[:, 1:2]
    p2 = p8[:, 2:3]
    pn = (p0 * p0 + p1 * p1) + p2 * p2
    d2 = (pn + cn_ref[...]) - 2.0 * r
    mn = jnp.min(d2, axis=1, keepdims=True)
    liota = lax.broadcasted_iota(jnp.int32, d2.shape, 1).astype(jnp.float32)
    sblk = jnp.min(
        jnp.where(d2 == mn, liota, jnp.float32(3.0e7)), axis=1, keepdims=True
    )
    s_ref[...] = sblk.astype(jnp.int32).reshape(1, 1, sblk.shape[0])
    oh = jnp.where(liota == sblk, 1.0, 0.0).astype(jnp.float32)
    u8 = p8 - jnp.dot(oh, e8_ref[...], preferred_element_type=jnp.float32)
    t = jnp.maximum(
        jnp.dot(x_ref[...], wx_ref[...], preferred_element_type=jnp.float32)
        + jnp.dot(u8, wd8_ref[...], preferred_element_type=jnp.float32),
        0.0,
    )
    msg_ref[...] = (
        jnp.dot(t, w2_ref[...], preferred_element_type=jnp.float32) + b2_ref[...]
    )


def _knn_msg_pallas(p8, x, ct8, cn, e8, wx, wd8, w2, b2, bn=400):
    n, d = x.shape
    grid = n // bn
    mp = ct8.shape[1]
    s3, msg = pl.pallas_call(
        _knn_msg_body,
        grid=(grid,),
        in_specs=[
            pl.BlockSpec((bn, 8), lambda i: (i, 0)),
            pl.BlockSpec((bn, d), lambda i: (i, 0)),
            pl.BlockSpec((8, mp), lambda i: (0, 0)),
            pl.BlockSpec((1, mp), lambda i: (0, 0)),
            pl.BlockSpec((mp, 8), lambda i: (0, 0)),
            pl.BlockSpec((d, d), lambda i: (0, 0)),
            pl.BlockSpec((8, d), lambda i: (0, 0)),
            pl.BlockSpec((d, d), lambda i: (0, 0)),
            pl.BlockSpec((1, d), lambda i: (0, 0)),
        ],
        out_specs=[
            pl.BlockSpec((1, 1, bn), lambda i: (i, 0, 0)),
            pl.BlockSpec((bn, d), lambda i: (i, 0)),
        ],
        out_shape=[
            jax.ShapeDtypeStruct((grid, 1, bn), jnp.int32),
            jax.ShapeDtypeStruct((n, d), jnp.float32),
        ],
    )(p8, x, ct8, cn, e8, wx, wd8, w2, b2)
    return s3.reshape(n), msg


def _sc_segmax(msg, s, m):
    n, d = msg.shape
    rpg = -(-(n // 4) // 8) * 8
    starts = [((n // 4) * g // 8) * 8 for g in range(4)]
    assert all(st + rpg <= n for st in starts) and (starts[3] + rpg == n)
    mesh = plsc.VectorSubcoreMesh(core_axis_name="c", subcore_axis_name="s")

    @functools.partial(
        pl.kernel,
        mesh=mesh,
        out_type=jax.ShapeDtypeStruct((2, m, d), jnp.float32),
        compiler_params=pltpu.CompilerParams(use_tc_tiling_on_sc=False),
        scratch_types=[
            pltpu.VMEM((m, 16), jnp.float32),
            pltpu.VMEM((rpg, 16), jnp.float32),
            pltpu.VMEM((rpg,), jnp.int32),
            pltpu.VMEM_SHARED((8, m, 16), jnp.float32),
        ],
    )
    def sk(msg_hbm, s_hbm, out_hbm, acc, msgv, sv, shared):
        c = lax.axis_index("c")
        sc = lax.axis_index("s")
        cg = sc % 8
        rgl = sc // 8
        col0 = cg * 16
        rg = c * 2 + rgl
        st = (rg * (n // 4)) // 8 * 8
        row0 = pl.multiple_of(st, 8)
        pltpu.sync_copy(s_hbm.at[pl.ds(row0, rpg)], sv)
        pltpu.sync_copy(msg_hbm.at[pl.ds(row0, rpg), pl.ds(col0, 16)], msgv)
        ninf = jnp.full((16,), _NEG_INF, jnp.float32)

        def init(i, _):
            acc[i] = ninf
            return 0

        lax.fori_loop(0, m, init, 0, unroll=8)

        nch = (rpg + 15) // 16
        last = ((rpg - 16) // 8) * 8

        def body(ch, _):
            base = pl.multiple_of(jnp.minimum(ch * 16, last), 8)
            svec = sv[pl.ds(base, 16)]
            for k in range(16):
                si = svec[k]
                acc[si] = jnp.maximum(acc[si], msgv[base + k])
            return 0

        lax.fori_loop(0, nch, body, 0)

        @pl.when(rgl == 1)
        def _():
            pltpu.sync_copy(acc, shared.at[cg])

        plsc.subcore_barrier()

        @pl.when(rgl == 0)
        def _():
            pltpu.sync_copy(shared.at[cg], msgv.at[pl.ds(0, m), :])

            def comb(i, _):
                acc[i] = jnp.maximum(acc[i], msgv[i])
                return 0

            lax.fori_loop(0, m, comb, 0, unroll=8)
            pltpu.sync_copy(acc, out_hbm.at[c, :, pl.ds(col0, 16)])

    return sk(msg, s)


def _mlpg_body(xd_ref, parts_ref, w1a_ref, w1b_ref, b1_ref, w2_ref,
               b2_ref, out_ref):
    agg = jnp.maximum(parts_ref[0], parts_ref[1])
    agg = jnp.where(agg == _NEG_INF, 0.0, agg)
    xd = xd_ref[...]
    t = jnp.maximum(
        jnp.dot(xd, w1a_ref[...], preferred_element_type=jnp.float32)
        + jnp.dot(agg, w1b_ref[...], preferred_element_type=jnp.float32)
        + b1_ref[...],
        0.0,
    )
    out_ref[...] = (
        xd
        + jnp.dot(t, w2_ref[...], preferred_element_type=jnp.float32)
        + b2_ref[...]
    )


def _mlpg_pallas(x_dst, parts, w1a, w1b, b1, w2, b2):
    m, d = x_dst.shape
    return pl.pallas_call(
        _mlpg_body,
        out_shape=jax.ShapeDtypeStruct((m, d), jnp.float32),
    )(x_dst, parts, w1a, w1b, b1.reshape(1, -1), w2, b2.reshape(1, -1))


def kernel(x, pos, edge_index, h_w1, h_b1, h_w2, h_b2, f_w1, f_b1, f_w2, f_b2,
           g_w1, g_b1, g_w2, g_b2):
    n, d = x.shape
    m = int(round(n * _RATIO))
    mp = 2560

    idx8, cx8, cy8, cz8, cn8 = _fps_pallas(pos, m)
    idx = idx8.reshape(-1)[:m]
    cxr = cx8.reshape(1, -1)
    cyr = cy8.reshape(1, -1)
    czr = cz8.reshape(1, -1)
    pos_s = jnp.concatenate([cxr, cyr, czr], axis=0).T[:m]

    ct8 = jnp.concatenate(
        [cxr, cyr, czr, jnp.zeros((5, mp), jnp.float32)], axis=0)
    cn = cn8.reshape(1, -1)

    idx_p = jnp.pad(idx, (0, mp - m))
    x_dst_p = _sc_gather(x, idx_p, mp)
    x_dst = x_dst_p[:m]

    e3 = _mlph_pallas(x_dst, pos_s, h_w1, h_b1, h_w2, h_b2)
    e8 = jnp.pad(e3, ((0, mp - m), (0, 5)))

    p8 = jnp.concatenate(
        [pos, jnp.ones((n, 1), jnp.float32), jnp.zeros((n, 4), jnp.float32)],
        axis=1)
    wd8 = jnp.concatenate([f_w1[:3], f_b1.reshape(1, -1),
                           jnp.zeros((4, d), jnp.float32)], axis=0)

    s, msg = _knn_msg_pallas(p8, x, ct8, cn, e8, f_w1[3:], wd8, f_w2,
                             f_b2.reshape(1, -1))

    parts = _sc_segmax(msg, s, m)

    out1 = _mlpg_pallas(x_dst, parts, g_w1[:d], g_w1[d:], g_b1, g_w2, g_b2)

    q = jnp.arange(n, dtype=jnp.int32)
    assign_index = jnp.stack([q, s])
    return (out1, pos_s, assign_index)
